# baseline (speedup 1.0000x reference)
import functools

import jax
import jax.numpy as jnp
from jax import lax
from jax.experimental import pallas as pl
from jax.experimental.pallas import tpu as pltpu
from jax.experimental.pallas import tpu_sc as plsc

F = 26
V = 100000
D = 32
B = 16384
NC = 2
L = 16
ICH = 4096
NCB = B // ICH
NG = F * NCB
NR = 2


def _sc_encode():
    mesh = plsc.VectorSubcoreMesh(core_axis_name="c", subcore_axis_name="s")

    @functools.partial(
        pl.kernel,
        out_type=jax.ShapeDtypeStruct((D, B), jnp.float32),
        mesh=mesh,
        scratch_types=[
            pltpu.VMEM((V,), jnp.float32),
            pltpu.VMEM((NR, ICH), jnp.int32),
            pltpu.VMEM((B,), jnp.float32),
            pltpu.SemaphoreType.DMA,
            pltpu.SemaphoreType.DMA((NR,)),
        ],
        compiler_params=pltpu.CompilerParams(needs_layout_passes=False),
    )
    def body(tbl_hbm, idx_hbm, out_hbm, rowbuf, ibuf, acc, rsem, isem):
        d = lax.axis_index("s") * NC + lax.axis_index("c")

        def fire_idx(g):
            pltpu.async_copy(
                idx_hbm.at[g // NCB, g % NCB], ibuf.at[g % NR], isem.at[g % NR]
            )

        def wait_idx(g):
            pltpu.make_async_copy(
                idx_hbm.at[g // NCB, g % NCB], ibuf.at[g % NR], isem.at[g % NR]
            ).wait()

        def fire_row(f):
            pltpu.async_copy(tbl_hbm.at[f, d], rowbuf, rsem)

        for g in range(NR - 1):
            fire_idx(g)
        fire_row(0)

        def run_field(f, first):
            pltpu.make_async_copy(tbl_hbm.at[f, d], rowbuf, rsem).wait()

            def chunk(cb, carry):
                g = f * NCB + cb
                wait_idx(g)
                slot = g % NR

                @plsc.parallel_loop(0, ICH // L, unroll=16)
                def _(i):
                    vec = ibuf[slot, pl.ds(i * L, L)]
                    vals = plsc.load_gather(rowbuf, [vec])
                    dst = acc.at[pl.ds(cb * ICH + i * L, L)]
                    if first:
                        dst[...] = vals
                    else:
                        plsc.addupdate(dst, vals)

                gn = g + NR - 1

                @pl.when(gn < NG)
                def _():
                    fire_idx(gn)

                return carry

            lax.fori_loop(0, NCB, chunk, 0)

            @pl.when(f < F - 1)
            def _():
                fire_row(f + 1)

        run_field(0, True)
        lax.fori_loop(1, F, lambda f, c: (run_field(f, False), c)[1], 0)
        pltpu.sync_copy(acc, out_hbm.at[d])

    return body


def kernel(tables, values):
    tbl_t = jnp.transpose(tables, (0, 2, 1))
    idx_prep = values.astype(jnp.int32).T.reshape(F, NCB, ICH)
    out_t = _sc_encode()(tbl_t, idx_prep)
    return out_t.T


# speedup vs baseline: 1.2240x; 1.2240x over previous
import functools

import jax
import jax.numpy as jnp
from jax import lax
from jax.experimental import pallas as pl
from jax.experimental.pallas import tpu as pltpu
from jax.experimental.pallas import tpu_sc as plsc

F = 26
V = 100000
D = 32
B = 16384
NC = 2
L = 16
ICH = 2048
NCB = B // ICH
NG = F * NCB
NR = 4


def _sc_encode():
    mesh = plsc.VectorSubcoreMesh(core_axis_name="c", subcore_axis_name="s")

    @functools.partial(
        pl.kernel,
        out_type=jax.ShapeDtypeStruct((D, B), jnp.float32),
        mesh=mesh,
        scratch_types=[
            pltpu.VMEM((V,), jnp.float32),
            pltpu.VMEM((NR, ICH), jnp.int32),
            pltpu.VMEM((B,), jnp.float32),
            pltpu.SemaphoreType.DMA,
            pltpu.SemaphoreType.DMA((NR,)),
        ],
        compiler_params=pltpu.CompilerParams(needs_layout_passes=False),
    )
    def body(tbl_hbm, idx_hbm, out_hbm, rowbuf, ibuf, acc, rsem, isem):
        d = lax.axis_index("s") * NC + lax.axis_index("c")

        def fire_idx(g):
            pltpu.async_copy(
                idx_hbm.at[g // NCB, g % NCB], ibuf.at[g % NR], isem.at[g % NR]
            )

        def wait_idx(g):
            pltpu.make_async_copy(
                idx_hbm.at[g // NCB, g % NCB], ibuf.at[g % NR], isem.at[g % NR]
            ).wait()

        def fire_row(f):
            pltpu.async_copy(tbl_hbm.at[f, d], rowbuf, rsem)

        for g in range(NR - 1):
            fire_idx(g)
        fire_row(0)

        def run_field(f, first):
            pltpu.make_async_copy(tbl_hbm.at[f, d], rowbuf, rsem).wait()

            def chunk(cb, carry):
                g = f * NCB + cb
                wait_idx(g)
                slot = g % NR

                @plsc.parallel_loop(0, ICH // L, unroll=16)
                def _(i):
                    vec = ibuf[slot, pl.ds(i * L, L)]
                    vals = plsc.load_gather(rowbuf, [vec])
                    dst = acc.at[pl.ds(cb * ICH + i * L, L)]
                    if first:
                        dst[...] = vals
                    else:
                        plsc.addupdate(dst, vals)

                gn = g + NR - 1

                @pl.when(gn < NG)
                def _():
                    fire_idx(gn)

                return carry

            lax.fori_loop(0, NCB, chunk, 0)

            @pl.when(f < F - 1)
            def _():
                fire_row(f + 1)

        run_field(0, True)
        lax.fori_loop(1, F, lambda f, c: (run_field(f, False), c)[1], 0)
        pltpu.sync_copy(acc, out_hbm.at[d])

    return body


def kernel(tables, values):
    tbl_t = jnp.transpose(tables, (0, 2, 1))
    idx_prep = values.astype(jnp.int32).T.reshape(F, NCB, ICH)
    out_t = _sc_encode()(tbl_t, idx_prep)
    return out_t.T


# speedup vs baseline: 1.2244x; 1.0004x over previous
import functools

import jax
import jax.numpy as jnp
from jax import lax
from jax.experimental import pallas as pl
from jax.experimental.pallas import tpu as pltpu
from jax.experimental.pallas import tpu_sc as plsc

F = 26
V = 100000
D = 32
B = 16384
NC = 2
L = 16
ICH = 2048
NCB = B // ICH
NG = F * NCB
NR = 4


def _sc_encode():
    mesh = plsc.VectorSubcoreMesh(core_axis_name="c", subcore_axis_name="s")

    @functools.partial(
        pl.kernel,
        out_type=jax.ShapeDtypeStruct((D, B), jnp.float32),
        mesh=mesh,
        scratch_types=[
            pltpu.VMEM((V,), jnp.float32),
            pltpu.VMEM((NR, ICH), jnp.int32),
            pltpu.VMEM((B,), jnp.float32),
            pltpu.SemaphoreType.DMA,
            pltpu.SemaphoreType.DMA((NR,)),
        ],
        compiler_params=pltpu.CompilerParams(needs_layout_passes=False),
    )
    def body(tbl_hbm, idx_hbm, out_hbm, rowbuf, ibuf, acc, rsem, isem):
        d = lax.axis_index("c") * 16 + lax.axis_index("s")

        def fire_idx(g):
            pltpu.async_copy(
                idx_hbm.at[g // NCB, g % NCB], ibuf.at[g % NR], isem.at[g % NR]
            )

        def wait_idx(g):
            pltpu.make_async_copy(
                idx_hbm.at[g // NCB, g % NCB], ibuf.at[g % NR], isem.at[g % NR]
            ).wait()

        def fire_row(f):
            pltpu.async_copy(tbl_hbm.at[f, d], rowbuf, rsem)

        for g in range(NR - 1):
            fire_idx(g)
        fire_row(0)

        def run_field(f, first):
            pltpu.make_async_copy(tbl_hbm.at[f, d], rowbuf, rsem).wait()

            def chunk(cb, carry):
                g = f * NCB + cb
                wait_idx(g)
                slot = g % NR

                @plsc.parallel_loop(0, ICH // L, unroll=16)
                def _(i):
                    vec = ibuf[slot, pl.ds(i * L, L)]
                    vals = plsc.load_gather(rowbuf, [vec])
                    dst = acc.at[pl.ds(cb * ICH + i * L, L)]
                    if first:
                        dst[...] = vals
                    else:
                        plsc.addupdate(dst, vals)

                gn = g + NR - 1

                @pl.when(gn < NG)
                def _():
                    fire_idx(gn)

                return carry

            lax.fori_loop(0, NCB, chunk, 0)

            @pl.when(f < F - 1)
            def _():
                fire_row(f + 1)

        run_field(0, True)
        lax.fori_loop(1, F, lambda f, c: (run_field(f, False), c)[1], 0)
        pltpu.sync_copy(acc, out_hbm.at[d])

    return body


def kernel(tables, values):
    tbl_t = jnp.transpose(tables, (0, 2, 1))
    idx_prep = values.astype(jnp.int32).T.reshape(F, NCB, ICH)
    out_t = _sc_encode()(tbl_t, idx_prep)
    return out_t.T
